# Initial kernel scaffold; baseline (speedup 1.0000x reference)
import jax
import jax.numpy as jnp
from jax.experimental import pallas as pl


def kernel(x_list, edge_index, W_conv, b_conv, W_heads, b_heads):
    raise NotImplementedError("write your pallas kernel here")


# speedup vs baseline: 60.4730x; 60.4730x over previous
import functools

import jax
import jax.numpy as jnp
from jax import lax
from jax.experimental import pallas as pl
from jax.experimental.pallas import tpu as pltpu
from jax.experimental.pallas import tpu_sc as plsc

N_NODES = 50000
HIDDEN = 64
N_INT = 500
PHASES = 100

NC, NS, L = 2, 16, 16
NW = NC * NS
NPAD = 50176
SLICE = NPAD // NS
ROWS2D = NPAD // 128
CHUNK = 128
GROUP = 8
ROWS_PER_TILE = 200
E_PAD = NW * ROWS_PER_TILE * CHUNK
DUMMY = NPAD - 1

_mesh = plsc.VectorSubcoreMesh(core_axis_name="c", subcore_axis_name="s")


def _zero_fill(ref, n):
    def body(i, carry):
        ref[pl.ds(i * L, L)] = jnp.zeros((L,), jnp.float32)
        return carry
    lax.fori_loop(0, n // L, body, 0)


def _one_fill(ref, n):
    def body(i, carry):
        ref[pl.ds(i * L, L)] = jnp.ones((L,), jnp.float32)
        return carry
    lax.fori_loop(0, n // L, body, 0)


@functools.partial(
    pl.kernel,
    out_type=jax.ShapeDtypeStruct((NC * NPAD,), jnp.float32),
    mesh=_mesh,
    scratch_types=[
        pltpu.VMEM((GROUP, CHUNK), jnp.int32),
        pltpu.VMEM((CHUNK,), jnp.float32),
        pltpu.VMEM((SLICE,), jnp.float32),
        pltpu.VMEM_SHARED((NPAD,), jnp.float32),
    ],
)
def _sc_degree(dst_hbm, out_hbm, dbuf, ones_v, zbuf, deg_sh):
    cid = lax.axis_index("c")
    sid = lax.axis_index("s")
    wid = cid * NS + sid
    _zero_fill(zbuf, SLICE)
    _one_fill(ones_v, CHUNK)
    pltpu.sync_copy(zbuf, deg_sh.at[pl.ds(sid * SLICE, SLICE)])
    plsc.subcore_barrier()
    base_row = wid * ROWS_PER_TILE

    def grp(gi, carry):
        pltpu.sync_copy(dst_hbm.at[pl.ds(base_row + gi * GROUP, GROUP)], dbuf)
        for j in range(GROUP):
            pltpu.sync_copy(ones_v, deg_sh.at[dbuf.at[j]], add=True)
        return carry

    lax.fori_loop(0, ROWS_PER_TILE // GROUP, grp, 0)
    plsc.subcore_barrier()
    pltpu.sync_copy(deg_sh.at[pl.ds(sid * SLICE, SLICE)], zbuf)
    pltpu.sync_copy(zbuf, out_hbm.at[pl.ds(cid * NPAD + sid * SLICE, SLICE)])


@functools.partial(
    pl.kernel,
    out_type=jax.ShapeDtypeStruct((NC * NPAD,), jnp.float32),
    mesh=_mesh,
    scratch_types=[
        pltpu.VMEM((GROUP, CHUNK), jnp.int32),
        pltpu.VMEM((GROUP, CHUNK), jnp.int32),
        pltpu.VMEM((CHUNK,), jnp.float32),
        pltpu.VMEM((SLICE,), jnp.float32),
        pltpu.VMEM_SHARED((NPAD,), jnp.float32),
        pltpu.VMEM_SHARED((NPAD,), jnp.float32),
    ],
)
def _sc_edge_sum(src_hbm, dst_hbm, g_hbm, out_hbm,
                 sbuf, dbuf, vals, zbuf, g_sh, acc_sh):
    cid = lax.axis_index("c")
    sid = lax.axis_index("s")
    wid = cid * NS + sid
    _zero_fill(zbuf, SLICE)
    pltpu.sync_copy(zbuf, acc_sh.at[pl.ds(sid * SLICE, SLICE)])
    pltpu.sync_copy(g_hbm.at[pl.ds(sid * SLICE, SLICE)], zbuf)
    pltpu.sync_copy(zbuf, g_sh.at[pl.ds(sid * SLICE, SLICE)])
    plsc.subcore_barrier()
    base_row = wid * ROWS_PER_TILE

    def grp(gi, carry):
        row = base_row + gi * GROUP
        pltpu.sync_copy(src_hbm.at[pl.ds(row, GROUP)], sbuf)
        pltpu.sync_copy(dst_hbm.at[pl.ds(row, GROUP)], dbuf)
        for j in range(GROUP):
            pltpu.sync_copy(g_sh.at[sbuf.at[j]], vals)
            pltpu.sync_copy(vals, acc_sh.at[dbuf.at[j]], add=True)
        return carry

    lax.fori_loop(0, ROWS_PER_TILE // GROUP, grp, 0)
    plsc.subcore_barrier()
    pltpu.sync_copy(acc_sh.at[pl.ds(sid * SLICE, SLICE)], zbuf)
    pltpu.sync_copy(zbuf, out_hbm.at[pl.ds(cid * NPAD + sid * SLICE, SLICE)])


def _tc_elem_body(degp_ref, x_ref, g_ref, dinv_ref):
    deg = degp_ref[0] + degp_ref[1] + 1.0
    dinv = lax.rsqrt(deg)
    dinv_ref[...] = dinv
    g_ref[...] = x_ref[...] * dinv


def _tc_elem(degp, xpad):
    return pl.pallas_call(
        _tc_elem_body,
        out_shape=(jax.ShapeDtypeStruct((ROWS2D, 128), jnp.float32),
                   jax.ShapeDtypeStruct((ROWS2D, 128), jnp.float32)),
    )(degp, xpad)


BI = 10
GI = N_INT // BI


def _tc_heads_body(acc_ref, g_ref, dinv_ref, w_ref, bc_ref, wh_ref, bh_ref,
                   out_ref):
    c = dinv_ref[0] * (acc_ref[0, 0] + acc_ref[1, 0] + g_ref[0])
    pre = c[:, :, None] * w_ref[0][None, None, :] + bc_ref[0][None, None, :]
    pre = jnp.maximum(pre, 0.0)
    out = lax.dot_general(pre, wh_ref[0], (((2,), (1,)), ((0,), (0,))),
                          preferred_element_type=jnp.float32)
    out_ref[0] = out + bh_ref[0][:, None, :]


def _tc_heads(acc3, g3, dinv3, w_row, bc2, W_heads, b_heads):
    acc4 = acc3.reshape(NC, GI, BI, PHASES)
    g4 = g3.reshape(GI, BI, PHASES)
    dinv4 = dinv3.reshape(GI, BI, PHASES)
    wh4 = W_heads.reshape(GI, BI, HIDDEN, PHASES)
    bh4 = b_heads.reshape(GI, BI, PHASES)
    out = pl.pallas_call(
        _tc_heads_body,
        grid=(GI,),
        in_specs=[
            pl.BlockSpec((NC, 1, BI, PHASES), lambda i: (0, i, 0, 0)),
            pl.BlockSpec((1, BI, PHASES), lambda i: (i, 0, 0)),
            pl.BlockSpec((1, BI, PHASES), lambda i: (i, 0, 0)),
            pl.BlockSpec((1, HIDDEN), lambda i: (0, 0)),
            pl.BlockSpec((1, HIDDEN), lambda i: (0, 0)),
            pl.BlockSpec((1, BI, HIDDEN, PHASES), lambda i: (i, 0, 0, 0)),
            pl.BlockSpec((1, BI, PHASES), lambda i: (i, 0, 0)),
        ],
        out_specs=pl.BlockSpec((1, BI, PHASES, PHASES), lambda i: (i, 0, 0, 0)),
        out_shape=jax.ShapeDtypeStruct((GI, BI, PHASES, PHASES), jnp.float32),
    )(acc4, g4, dinv4, w_row, bc2, wh4, bh4)
    return out.reshape(N_INT, PHASES, PHASES)


def kernel(x_list, edge_index, W_conv, b_conv, W_heads, b_heads):
    n = x_list.size
    e = edge_index.shape[1]
    src = edge_index[0]
    dst = edge_index[1]
    pad_e = jnp.full((E_PAD - e,), DUMMY, jnp.int32)
    src2d = jnp.concatenate([src, pad_e]).reshape(-1, CHUNK)
    dst2d = jnp.concatenate([dst, pad_e]).reshape(-1, CHUNK)
    xpad = jnp.pad(x_list.reshape(-1), (0, NPAD - n)).reshape(ROWS2D, 128)

    degp = _sc_degree(dst2d).reshape(NC, NPAD)
    g2, dinv2 = _tc_elem(degp.reshape(NC, ROWS2D, 128), xpad)
    g = g2.reshape(-1)
    accp = _sc_edge_sum(src2d, dst2d, g).reshape(NC, NPAD)

    acc3 = accp[:, :n].reshape(NC, N_INT, PHASES)
    g3 = g[:n].reshape(N_INT, PHASES)
    dinv3 = dinv2.reshape(-1)[:n].reshape(N_INT, PHASES)
    w_row = W_conv.reshape(1, HIDDEN)
    bc2 = b_conv.reshape(1, HIDDEN)
    return _tc_heads(acc3, g3, dinv3, w_row, bc2, W_heads, b_heads)


# speedup vs baseline: 79.9367x; 1.3219x over previous
import functools

import jax
import jax.numpy as jnp
from jax import lax
from jax.experimental import pallas as pl
from jax.experimental.pallas import tpu as pltpu
from jax.experimental.pallas import tpu_sc as plsc

N_NODES = 50000
HIDDEN = 64
N_INT = 500
PHASES = 100

NC, NS, L = 2, 16, 16
NW = NC * NS
NPAD = 50176
SLICE = NPAD // NS
ROWS2D = NPAD // 128
CHUNK = 128
E_ROWS = 6250
RPT = 200
LAST_FULL = 48
BATCH = 8

_mesh = plsc.VectorSubcoreMesh(core_axis_name="c", subcore_axis_name="s")


def _zero_fill(ref, n):
    def body(i, carry):
        ref[pl.ds(i * L, L)] = jnp.zeros((L,), jnp.float32)
        return carry
    lax.fori_loop(0, n // L, body, 0)


def _one_fill(ref, n):
    def body(i, carry):
        ref[pl.ds(i * L, L)] = jnp.ones((L,), jnp.float32)
        return carry
    lax.fori_loop(0, n // L, body, 0)


def _load_rows(hbm, buf, wid):
    @pl.when(wid < NW - 1)
    def _():
        pltpu.sync_copy(hbm.at[pl.ds(wid * RPT, RPT)], buf)

    @pl.when(wid == NW - 1)
    def _():
        pltpu.sync_copy(hbm.at[pl.ds((NW - 1) * RPT, LAST_FULL)],
                        buf.at[pl.ds(0, LAST_FULL)])
        pltpu.sync_copy(hbm.at[pl.ds((NW - 1) * RPT + LAST_FULL, 2)],
                        buf.at[pl.ds(LAST_FULL, 2)])


@functools.partial(
    pl.kernel,
    out_type=jax.ShapeDtypeStruct((NC * NPAD,), jnp.float32),
    mesh=_mesh,
    scratch_types=[
        pltpu.VMEM((RPT, CHUNK), jnp.int32),
        pltpu.VMEM((CHUNK,), jnp.float32),
        pltpu.VMEM((SLICE,), jnp.float32),
        pltpu.VMEM_SHARED((NPAD,), jnp.float32),
        pltpu.SemaphoreType.DMA,
    ],
)
def _sc_degree(dst_hbm, out_hbm, dbuf, ones_v, zbuf, deg_sh, sem):
    cid = lax.axis_index("c")
    sid = lax.axis_index("s")
    wid = cid * NS + sid
    _zero_fill(zbuf, SLICE)
    _one_fill(ones_v, CHUNK)
    pltpu.sync_copy(zbuf, deg_sh.at[pl.ds(sid * SLICE, SLICE)])
    _load_rows(dst_hbm, dbuf, wid)
    plsc.subcore_barrier()

    ngroups = jnp.where(wid == NW - 1, LAST_FULL // BATCH, RPT // BATCH)

    def grp(gi, carry):
        cps = [pltpu.async_copy(ones_v, deg_sh.at[dbuf.at[gi * BATCH + j]],
                                sem, add=True) for j in range(BATCH)]
        for cp in cps:
            cp.wait()
        return carry

    lax.fori_loop(0, ngroups, grp, 0)

    @pl.when(wid == NW - 1)
    def _():
        cps = [pltpu.async_copy(ones_v, deg_sh.at[dbuf.at[LAST_FULL + j]],
                                sem, add=True) for j in range(2)]
        for cp in cps:
            cp.wait()

    plsc.subcore_barrier()
    pltpu.sync_copy(deg_sh.at[pl.ds(sid * SLICE, SLICE)], zbuf)
    pltpu.sync_copy(zbuf, out_hbm.at[pl.ds(cid * NPAD + sid * SLICE, SLICE)])


@functools.partial(
    pl.kernel,
    out_type=jax.ShapeDtypeStruct((NC * NPAD,), jnp.float32),
    mesh=_mesh,
    scratch_types=[
        pltpu.VMEM((RPT, CHUNK), jnp.int32),
        pltpu.VMEM((RPT, CHUNK), jnp.int32),
        pltpu.VMEM((BATCH, CHUNK), jnp.float32),
        pltpu.VMEM((SLICE,), jnp.float32),
        pltpu.VMEM_SHARED((NPAD,), jnp.float32),
        pltpu.VMEM_SHARED((NPAD,), jnp.float32),
        pltpu.SemaphoreType.DMA,
        pltpu.SemaphoreType.DMA,
    ],
)
def _sc_edge_sum(src_hbm, dst_hbm, g_hbm, out_hbm,
                 sbuf, dbuf, vals, zbuf, g_sh, acc_sh, gsem, ssem):
    cid = lax.axis_index("c")
    sid = lax.axis_index("s")
    wid = cid * NS + sid
    _zero_fill(zbuf, SLICE)
    pltpu.sync_copy(zbuf, acc_sh.at[pl.ds(sid * SLICE, SLICE)])
    pltpu.sync_copy(g_hbm.at[pl.ds(sid * SLICE, SLICE)], zbuf)
    pltpu.sync_copy(zbuf, g_sh.at[pl.ds(sid * SLICE, SLICE)])
    _load_rows(src_hbm, sbuf, wid)
    _load_rows(dst_hbm, dbuf, wid)
    plsc.subcore_barrier()

    ngroups = jnp.where(wid == NW - 1, LAST_FULL // BATCH, RPT // BATCH)

    def do_batch(base, nrows):
        gcps = [pltpu.async_copy(g_sh.at[sbuf.at[base + j]], vals.at[j], gsem)
                for j in range(nrows)]
        for cp in gcps:
            cp.wait()
        scps = [pltpu.async_copy(vals.at[j], acc_sh.at[dbuf.at[base + j]],
                                 ssem, add=True) for j in range(nrows)]
        for cp in scps:
            cp.wait()

    def grp(gi, carry):
        do_batch(gi * BATCH, BATCH)
        return carry

    lax.fori_loop(0, ngroups, grp, 0)

    @pl.when(wid == NW - 1)
    def _():
        do_batch(LAST_FULL, 2)

    plsc.subcore_barrier()
    pltpu.sync_copy(acc_sh.at[pl.ds(sid * SLICE, SLICE)], zbuf)
    pltpu.sync_copy(zbuf, out_hbm.at[pl.ds(cid * NPAD + sid * SLICE, SLICE)])


def _tc_elem_body(degp_ref, x_ref, g_ref, dinv_ref):
    deg = degp_ref[0] + degp_ref[1] + 1.0
    dinv = lax.rsqrt(deg)
    dinv_ref[...] = dinv
    g_ref[...] = x_ref[...] * dinv


def _tc_elem(degp, xpad):
    return pl.pallas_call(
        _tc_elem_body,
        out_shape=(jax.ShapeDtypeStruct((ROWS2D, 128), jnp.float32),
                   jax.ShapeDtypeStruct((ROWS2D, 128), jnp.float32)),
    )(degp, xpad)


BI = 10
GI = N_INT // BI


def _tc_heads_body(acc_ref, g_ref, dinv_ref, w_ref, bc_ref, wh_ref, bh_ref,
                   out_ref):
    c = dinv_ref[0] * (acc_ref[0, 0] + acc_ref[1, 0] + g_ref[0])
    pre = c[:, :, None] * w_ref[0][None, None, :] + bc_ref[0][None, None, :]
    pre = jnp.maximum(pre, 0.0)
    out = lax.dot_general(pre, wh_ref[0], (((2,), (1,)), ((0,), (0,))),
                          preferred_element_type=jnp.float32)
    out_ref[0] = out + bh_ref[0][:, None, :]


def _tc_heads(acc3, g3, dinv3, w_row, bc2, W_heads, b_heads):
    acc4 = acc3.reshape(NC, GI, BI, PHASES)
    g4 = g3.reshape(GI, BI, PHASES)
    dinv4 = dinv3.reshape(GI, BI, PHASES)
    wh4 = W_heads.reshape(GI, BI, HIDDEN, PHASES)
    bh4 = b_heads.reshape(GI, BI, PHASES)
    out = pl.pallas_call(
        _tc_heads_body,
        grid=(GI,),
        in_specs=[
            pl.BlockSpec((NC, 1, BI, PHASES), lambda i: (0, i, 0, 0)),
            pl.BlockSpec((1, BI, PHASES), lambda i: (i, 0, 0)),
            pl.BlockSpec((1, BI, PHASES), lambda i: (i, 0, 0)),
            pl.BlockSpec((1, HIDDEN), lambda i: (0, 0)),
            pl.BlockSpec((1, HIDDEN), lambda i: (0, 0)),
            pl.BlockSpec((1, BI, HIDDEN, PHASES), lambda i: (i, 0, 0, 0)),
            pl.BlockSpec((1, BI, PHASES), lambda i: (i, 0, 0)),
        ],
        out_specs=pl.BlockSpec((1, BI, PHASES, PHASES), lambda i: (i, 0, 0, 0)),
        out_shape=jax.ShapeDtypeStruct((GI, BI, PHASES, PHASES), jnp.float32),
    )(acc4, g4, dinv4, w_row, bc2, wh4, bh4)
    return out.reshape(N_INT, PHASES, PHASES)


def kernel(x_list, edge_index, W_conv, b_conv, W_heads, b_heads):
    n = x_list.size
    src2d = edge_index[0].reshape(E_ROWS, CHUNK)
    dst2d = edge_index[1].reshape(E_ROWS, CHUNK)
    xpad = jnp.pad(x_list.reshape(-1), (0, NPAD - n)).reshape(ROWS2D, 128)

    degp = _sc_degree(dst2d).reshape(NC, NPAD)
    g2, dinv2 = _tc_elem(degp.reshape(NC, ROWS2D, 128), xpad)
    g = g2.reshape(-1)
    accp = _sc_edge_sum(src2d, dst2d, g).reshape(NC, NPAD)

    acc3 = accp[:, :n].reshape(NC, N_INT, PHASES)
    g3 = g[:n].reshape(N_INT, PHASES)
    dinv3 = dinv2.reshape(-1)[:n].reshape(N_INT, PHASES)
    w_row = W_conv.reshape(1, HIDDEN)
    bc2 = b_conv.reshape(1, HIDDEN)
    return _tc_heads(acc3, g3, dinv3, w_row, bc2, W_heads, b_heads)
